# Initial kernel scaffold; baseline (speedup 1.0000x reference)
import jax
import jax.numpy as jnp
from jax.experimental import pallas as pl


def kernel(idx, targets, table):
    raise NotImplementedError("write your pallas kernel here")


# speedup vs baseline: 1.2287x; 1.2287x over previous
import functools

import jax
import jax.numpy as jnp
from jax import lax
from jax.experimental import pallas as pl
from jax.experimental.pallas import tpu as pltpu
from jax.experimental.pallas import tpu_sc as plsc

NC, NS, L = 2, 16, 16
NW = NC * NS
N_ROWS = 1024 * 50
D = 1000
RPW = N_ROWS // NW
CH = 64
NCHUNK = RPW // CH
UNROLL = 8


def _sc_body(table, idxr, tgtr, out, maxs, sums, tgtl,
             idx_v, tgt_v, rows_v, m_v, s_v, t_v, sem):
    wid = lax.axis_index("s") * NC + lax.axis_index("c")
    base = wid * RPW
    pltpu.sync_copy(idxr.at[pl.ds(base, RPW)], idx_v)
    pltpu.sync_copy(tgtr.at[pl.ds(base, RPW)], tgt_v)

    lanes = lax.iota(jnp.int32, L)
    neg = jnp.full((L,), -jnp.inf, jnp.float32)
    zero = jnp.zeros((L,), jnp.float32)

    def chunk_body(c, carry):
        r0 = c * CH
        pltpu.async_copy(table.at[idx_v.at[pl.ds(r0, CH)]], rows_v, sem).wait()

        for g in range(CH // L):
            rvec = lanes + g * L

            def max_body(j, accs):
                a0, a1, a2, a3 = accs
                vs = []
                for u in range(UNROLL):
                    jv = jnp.full((L,), j * UNROLL + u, jnp.int32)
                    vs.append(plsc.load_gather(rows_v, [rvec, jv]))
                a0 = jnp.maximum(a0, jnp.maximum(vs[0], vs[4]))
                a1 = jnp.maximum(a1, jnp.maximum(vs[1], vs[5]))
                a2 = jnp.maximum(a2, jnp.maximum(vs[2], vs[6]))
                a3 = jnp.maximum(a3, jnp.maximum(vs[3], vs[7]))
                return a0, a1, a2, a3

            a0, a1, a2, a3 = lax.fori_loop(
                0, D // UNROLL, max_body, (neg, neg, neg, neg))
            m = jnp.maximum(jnp.maximum(a0, a1), jnp.maximum(a2, a3))

            def sum_body(j, accs):
                s0, s1, s2, s3 = accs
                es = []
                for u in range(UNROLL):
                    jv = jnp.full((L,), j * UNROLL + u, jnp.int32)
                    es.append(jnp.exp(plsc.load_gather(rows_v, [rvec, jv]) - m))
                s0 = s0 + (es[0] + es[4])
                s1 = s1 + (es[1] + es[5])
                s2 = s2 + (es[2] + es[6])
                s3 = s3 + (es[3] + es[7])
                return s0, s1, s2, s3

            s0, s1, s2, s3 = lax.fori_loop(
                0, D // UNROLL, sum_body, (zero, zero, zero, zero))
            s = (s0 + s1) + (s2 + s3)

            m_v[pl.ds(r0 + g * L, L)] = m
            s_v[pl.ds(r0 + g * L, L)] = s
            tg = tgt_v[pl.ds(r0 + g * L, L)]
            t_v[pl.ds(r0 + g * L, L)] = plsc.load_gather(rows_v, [rvec, tg])

        pltpu.sync_copy(rows_v, out.at[pl.ds(base + r0, CH)])
        return carry

    lax.fori_loop(0, NCHUNK, chunk_body, 0)
    pltpu.sync_copy(m_v, maxs.at[pl.ds(base, RPW)])
    pltpu.sync_copy(s_v, sums.at[pl.ds(base, RPW)])
    pltpu.sync_copy(t_v, tgtl.at[pl.ds(base, RPW)])


@functools.partial(jax.jit, static_argnums=())
def _sc_call(table, idxf, tgtf):
    mesh = plsc.VectorSubcoreMesh(core_axis_name="c", subcore_axis_name="s")
    return pl.kernel(
        _sc_body,
        out_type=(
            jax.ShapeDtypeStruct((N_ROWS, D), jnp.float32),
            jax.ShapeDtypeStruct((N_ROWS,), jnp.float32),
            jax.ShapeDtypeStruct((N_ROWS,), jnp.float32),
            jax.ShapeDtypeStruct((N_ROWS,), jnp.float32),
        ),
        mesh=mesh,
        compiler_params=pltpu.CompilerParams(
            use_tc_tiling_on_sc=False, needs_layout_passes=False),
        scratch_types=[
            pltpu.VMEM((RPW,), jnp.int32),
            pltpu.VMEM((RPW,), jnp.int32),
            pltpu.VMEM((CH, D), jnp.float32),
            pltpu.VMEM((RPW,), jnp.float32),
            pltpu.VMEM((RPW,), jnp.float32),
            pltpu.VMEM((RPW,), jnp.float32),
            pltpu.SemaphoreType.DMA,
        ],
    )(table, idxf, tgtf)


def _loss_body(m_ref, s_ref, t_ref, o_ref):
    nll = jnp.log(s_ref[...]) + m_ref[...] - t_ref[...]
    o_ref[0, 0] = jnp.sum(nll) * (1.0 / N_ROWS)


def _loss_call(maxs, sums, tgtl):
    r = pl.pallas_call(
        _loss_body,
        out_shape=jax.ShapeDtypeStruct((1, 1), jnp.float32),
        out_specs=pl.BlockSpec(memory_space=pltpu.SMEM),
    )(maxs.reshape(400, 128), sums.reshape(400, 128), tgtl.reshape(400, 128))
    return r[0, 0]


def kernel(idx, targets, table):
    idxf = idx.reshape(-1).astype(jnp.int32)
    tgtf = targets.reshape(-1).astype(jnp.int32)
    out, maxs, sums, tgtl = _sc_call(table, idxf, tgtf)
    loss = _loss_call(maxs, sums, tgtl)
    return out, loss


# speedup vs baseline: 1.5508x; 1.2622x over previous
import functools

import jax
import jax.numpy as jnp
from jax import lax
from jax.experimental import pallas as pl
from jax.experimental.pallas import tpu as pltpu
from jax.experimental.pallas import tpu_sc as plsc

NC, NS, L = 2, 16, 16
NW = NC * NS
N_ROWS = 1024 * 50
D = 1000
RPW = N_ROWS // NW
CH = 16
NCHUNK = RPW // CH
NBUF = 4
UNROLL = 8


def _sc_body(table, idxr, tgtr, out, maxs, sums, tgtl,
             idx_v, tgt_v, rows_v, m_v, s_v, t_v, gsem, osem):
    wid = lax.axis_index("s") * NC + lax.axis_index("c")
    base = wid * RPW
    pltpu.sync_copy(idxr.at[pl.ds(base, RPW)], idx_v)
    pltpu.sync_copy(tgtr.at[pl.ds(base, RPW)], tgt_v)

    lanes = lax.iota(jnp.int32, L)
    neg = jnp.full((L,), -jnp.inf, jnp.float32)
    zero = jnp.zeros((L,), jnp.float32)

    def gather(c, b):
        pltpu.make_async_copy(
            table.at[idx_v.at[pl.ds(c * CH, CH)]], rows_v.at[b], gsem.at[b]
        ).start()

    def compute(c, b):
        rb = rows_v.at[b]

        def max_body(j, accs):
            a0, a1, a2, a3 = accs
            vs = []
            for u in range(UNROLL):
                jv = jnp.full((L,), j * UNROLL + u, jnp.int32)
                vs.append(plsc.load_gather(rb, [lanes, jv]))
            a0 = jnp.maximum(a0, jnp.maximum(vs[0], vs[4]))
            a1 = jnp.maximum(a1, jnp.maximum(vs[1], vs[5]))
            a2 = jnp.maximum(a2, jnp.maximum(vs[2], vs[6]))
            a3 = jnp.maximum(a3, jnp.maximum(vs[3], vs[7]))
            return a0, a1, a2, a3

        a0, a1, a2, a3 = lax.fori_loop(
            0, D // UNROLL, max_body, (neg, neg, neg, neg))
        m = jnp.maximum(jnp.maximum(a0, a1), jnp.maximum(a2, a3))

        def sum_body(j, accs):
            s0, s1, s2, s3 = accs
            es = []
            for u in range(UNROLL):
                jv = jnp.full((L,), j * UNROLL + u, jnp.int32)
                es.append(jnp.exp(plsc.load_gather(rb, [lanes, jv]) - m))
            s0 = s0 + (es[0] + es[4])
            s1 = s1 + (es[1] + es[5])
            s2 = s2 + (es[2] + es[6])
            s3 = s3 + (es[3] + es[7])
            return s0, s1, s2, s3

        s0, s1, s2, s3 = lax.fori_loop(
            0, D // UNROLL, sum_body, (zero, zero, zero, zero))
        s = (s0 + s1) + (s2 + s3)

        m_v[pl.ds(c * CH, L)] = m
        s_v[pl.ds(c * CH, L)] = s
        tg = tgt_v[pl.ds(c * CH, L)]
        t_v[pl.ds(c * CH, L)] = plsc.load_gather(rb, [lanes, tg])

    gather(0, 0)
    gather(1, 1)

    def loop_body(i, carry):
        for j in range(NBUF):
            c = NBUF * i + j
            b, b2 = j, (j + 2) % NBUF
            pltpu.make_async_copy(
                table.at[idx_v.at[pl.ds(c * CH, CH)]], rows_v.at[b], gsem.at[b]
            ).wait()
            compute(c, b)
            pltpu.make_async_copy(
                rows_v.at[b], out.at[pl.ds(base + c * CH, CH)], osem.at[b]
            ).start()
            @pl.when(c >= 2)
            def _():
                pltpu.make_async_copy(
                    rows_v.at[b2], out.at[pl.ds(base, CH)], osem.at[b2]
                ).wait()

            @pl.when(c + 2 < NCHUNK)
            def _():
                gather(c + 2, b2)

        return carry

    lax.fori_loop(0, NCHUNK // NBUF, loop_body, 0)
    for b in ((NCHUNK - 2) % NBUF, (NCHUNK - 1) % NBUF):
        pltpu.make_async_copy(
            rows_v.at[b], out.at[pl.ds(base, CH)], osem.at[b]
        ).wait()
    pltpu.sync_copy(m_v, maxs.at[pl.ds(base, RPW)])
    pltpu.sync_copy(s_v, sums.at[pl.ds(base, RPW)])
    pltpu.sync_copy(t_v, tgtl.at[pl.ds(base, RPW)])


@functools.partial(jax.jit, static_argnums=())
def _sc_call(table, idxf, tgtf):
    mesh = plsc.VectorSubcoreMesh(core_axis_name="c", subcore_axis_name="s")
    return pl.kernel(
        _sc_body,
        out_type=(
            jax.ShapeDtypeStruct((N_ROWS, D), jnp.float32),
            jax.ShapeDtypeStruct((N_ROWS,), jnp.float32),
            jax.ShapeDtypeStruct((N_ROWS,), jnp.float32),
            jax.ShapeDtypeStruct((N_ROWS,), jnp.float32),
        ),
        mesh=mesh,
        compiler_params=pltpu.CompilerParams(
            use_tc_tiling_on_sc=False, needs_layout_passes=False),
        scratch_types=[
            pltpu.VMEM((RPW,), jnp.int32),
            pltpu.VMEM((RPW,), jnp.int32),
            pltpu.VMEM((NBUF, CH, D), jnp.float32),
            pltpu.VMEM((RPW,), jnp.float32),
            pltpu.VMEM((RPW,), jnp.float32),
            pltpu.VMEM((RPW,), jnp.float32),
            pltpu.SemaphoreType.DMA((NBUF,)),
            pltpu.SemaphoreType.DMA((NBUF,)),
        ],
    )(table, idxf, tgtf)


def _loss_body(m_ref, s_ref, t_ref, o_ref):
    nll = jnp.log(s_ref[...]) + m_ref[...] - t_ref[...]
    o_ref[0, 0] = jnp.sum(nll) * (1.0 / N_ROWS)


def _loss_call(maxs, sums, tgtl):
    r = pl.pallas_call(
        _loss_body,
        out_shape=jax.ShapeDtypeStruct((1, 1), jnp.float32),
        out_specs=pl.BlockSpec(memory_space=pltpu.SMEM),
    )(maxs.reshape(400, 128), sums.reshape(400, 128), tgtl.reshape(400, 128))
    return r[0, 0]


def kernel(idx, targets, table):
    idxf = idx.reshape(-1).astype(jnp.int32)
    tgtf = targets.reshape(-1).astype(jnp.int32)
    out, maxs, sums, tgtl = _sc_call(table, idxf, tgtf)
    loss = _loss_call(maxs, sums, tgtl)
    return out, loss


# speedup vs baseline: 1.6296x; 1.0508x over previous
import functools

import jax
import jax.numpy as jnp
from jax import lax
from jax.experimental import pallas as pl
from jax.experimental.pallas import tpu as pltpu
from jax.experimental.pallas import tpu_sc as plsc

NC, NS, L = 2, 16, 16
NW = NC * NS
N_ROWS = 1024 * 50
D = 1000
RPW = N_ROWS // NW
CH = 16
NCHUNK = RPW // CH
NBUF = 4


def _rowstats_body(tab_ref, lse_ref):
    x = tab_ref[...]
    m = jnp.max(x, axis=1, keepdims=True)
    s = jnp.sum(jnp.exp(x - m), axis=1, keepdims=True)
    lse_ref[...] = jnp.log(s) + m


def _rowstats_call(table):
    r = pl.pallas_call(
        _rowstats_body,
        out_shape=jax.ShapeDtypeStruct((D, 1), jnp.float32),
    )(table)
    return r.reshape(D)


def _sc_body(table, rowlse, idxr, tgtr, out, lse_o, tgl_o,
             idx_v, tgt_v, rows_v, lse_v, t_v, gsem, lsem, osem):
    wid = lax.axis_index("s") * NC + lax.axis_index("c")
    base = wid * RPW
    pltpu.sync_copy(idxr.at[pl.ds(base, RPW)], idx_v)
    pltpu.sync_copy(tgtr.at[pl.ds(base, RPW)], tgt_v)

    lanes = lax.iota(jnp.int32, L)

    def gather(c, b):
        iv = idx_v.at[pl.ds(c * CH, CH)]
        pltpu.make_async_copy(table.at[iv], rows_v.at[b], gsem.at[b]).start()
        pltpu.make_async_copy(
            rowlse.at[iv], lse_v.at[pl.ds(c * CH, CH)], lsem.at[b]
        ).start()

    gather(0, 0)
    gather(1, 1)

    def loop_body(i, carry):
        for j in range(NBUF):
            c = NBUF * i + j
            b, b2 = j, (j + 2) % NBUF
            iv = idx_v.at[pl.ds(c * CH, CH)]
            pltpu.make_async_copy(table.at[iv], rows_v.at[b], gsem.at[b]).wait()
            pltpu.make_async_copy(
                rowlse.at[iv], lse_v.at[pl.ds(c * CH, CH)], lsem.at[b]
            ).wait()
            tg = tgt_v[pl.ds(c * CH, L)]
            t_v[pl.ds(c * CH, L)] = plsc.load_gather(rows_v.at[b], [lanes, tg])
            pltpu.make_async_copy(
                rows_v.at[b], out.at[pl.ds(base + c * CH, CH)], osem.at[b]
            ).start()
            @pl.when(c >= 2)
            def _():
                pltpu.make_async_copy(
                    rows_v.at[b2], out.at[pl.ds(base, CH)], osem.at[b2]
                ).wait()

            @pl.when(c + 2 < NCHUNK)
            def _():
                gather(c + 2, b2)

        return carry

    lax.fori_loop(0, NCHUNK // NBUF, loop_body, 0)
    for b in ((NCHUNK - 2) % NBUF, (NCHUNK - 1) % NBUF):
        pltpu.make_async_copy(
            rows_v.at[b], out.at[pl.ds(base, CH)], osem.at[b]
        ).wait()
    pltpu.sync_copy(lse_v, lse_o.at[pl.ds(base, RPW)])
    pltpu.sync_copy(t_v, tgl_o.at[pl.ds(base, RPW)])


@functools.partial(jax.jit, static_argnums=())
def _sc_call(table, rowlse, idxf, tgtf):
    mesh = plsc.VectorSubcoreMesh(core_axis_name="c", subcore_axis_name="s")
    return pl.kernel(
        _sc_body,
        out_type=(
            jax.ShapeDtypeStruct((N_ROWS, D), jnp.float32),
            jax.ShapeDtypeStruct((N_ROWS,), jnp.float32),
            jax.ShapeDtypeStruct((N_ROWS,), jnp.float32),
        ),
        mesh=mesh,
        compiler_params=pltpu.CompilerParams(
            use_tc_tiling_on_sc=False, needs_layout_passes=False),
        scratch_types=[
            pltpu.VMEM((RPW,), jnp.int32),
            pltpu.VMEM((RPW,), jnp.int32),
            pltpu.VMEM((NBUF, CH, D), jnp.float32),
            pltpu.VMEM((RPW,), jnp.float32),
            pltpu.VMEM((RPW,), jnp.float32),
            pltpu.SemaphoreType.DMA((NBUF,)),
            pltpu.SemaphoreType.DMA((NBUF,)),
            pltpu.SemaphoreType.DMA((NBUF,)),
        ],
    )(table, rowlse, idxf, tgtf)


def _loss_body(l_ref, t_ref, o_ref):
    o_ref[0, 0] = jnp.sum(l_ref[...] - t_ref[...]) * (1.0 / N_ROWS)


def _loss_call(lse, tgtl):
    r = pl.pallas_call(
        _loss_body,
        out_shape=jax.ShapeDtypeStruct((1, 1), jnp.float32),
        out_specs=pl.BlockSpec(memory_space=pltpu.SMEM),
    )(lse.reshape(400, 128), tgtl.reshape(400, 128))
    return r[0, 0]


def kernel(idx, targets, table):
    idxf = idx.reshape(-1).astype(jnp.int32)
    tgtf = targets.reshape(-1).astype(jnp.int32)
    rowlse = _rowstats_call(table)
    out, lse, tgtl = _sc_call(table, rowlse, idxf, tgtf)
    loss = _loss_call(lse, tgtl)
    return out, loss


# speedup vs baseline: 1.7065x; 1.0472x over previous
import functools

import jax
import jax.numpy as jnp
from jax import lax
from jax.experimental import pallas as pl
from jax.experimental.pallas import tpu as pltpu
from jax.experimental.pallas import tpu_sc as plsc

NC, NS, L = 2, 16, 16
NW = NC * NS
N_ROWS = 1024 * 50
D = 1000
DP = 1008
RPW = N_ROWS // NW
CH = 16
NCHUNK = RPW // CH
NBUF = 4


def _rowstats_body(tab_ref, lse_ref):
    x = tab_ref[...]
    m = jnp.max(x, axis=1, keepdims=True)
    s = jnp.sum(jnp.exp(x - m), axis=1, keepdims=True)
    lse_ref[...] = jnp.log(s) + m


def _rowstats_call(table):
    r = pl.pallas_call(
        _rowstats_body,
        out_shape=jax.ShapeDtypeStruct((D, 1), jnp.float32),
    )(table)
    return r.reshape(D)


def _sc_body(table, idxr, tgtr, out, lse_o, tgl_o,
             idx_v, tgt_v, rows_v, lse_v, t_v, gsem, osem):
    wid = lax.axis_index("s") * NC + lax.axis_index("c")
    base = wid * RPW
    pltpu.sync_copy(idxr.at[pl.ds(base, RPW)], idx_v)
    pltpu.sync_copy(tgtr.at[pl.ds(base, RPW)], tgt_v)

    lanes = lax.iota(jnp.int32, L)
    lse_col = jnp.full((L,), D, jnp.int32)

    def gather(c, b):
        iv = idx_v.at[pl.ds(c * CH, CH)]
        pltpu.make_async_copy(table.at[iv], rows_v.at[b], gsem.at[b]).start()

    gather(0, 0)
    gather(1, 1)

    def loop_body(i, carry):
        for j in range(NBUF):
            c = NBUF * i + j
            b, b2 = j, (j + 2) % NBUF
            iv = idx_v.at[pl.ds(c * CH, CH)]
            pltpu.make_async_copy(table.at[iv], rows_v.at[b], gsem.at[b]).wait()
            tg = tgt_v[pl.ds(c * CH, L)]
            t_v[pl.ds(c * CH, L)] = plsc.load_gather(rows_v.at[b], [lanes, tg])
            lse_v[pl.ds(c * CH, L)] = plsc.load_gather(
                rows_v.at[b], [lanes, lse_col])
            pltpu.make_async_copy(
                rows_v.at[b, :, pl.ds(0, D)],
                out.at[pl.ds(base + c * CH, CH)], osem.at[b]
            ).start()
            @pl.when(c >= 2)
            def _():
                pltpu.make_async_copy(
                    rows_v.at[b2, :, pl.ds(0, D)], out.at[pl.ds(base, CH)],
                    osem.at[b2]
                ).wait()

            @pl.when(c + 2 < NCHUNK)
            def _():
                gather(c + 2, b2)

        return carry

    lax.fori_loop(0, NCHUNK // NBUF, loop_body, 0)
    for b in ((NCHUNK - 2) % NBUF, (NCHUNK - 1) % NBUF):
        pltpu.make_async_copy(
            rows_v.at[b, :, pl.ds(0, D)], out.at[pl.ds(base, CH)], osem.at[b]
        ).wait()
    pltpu.sync_copy(lse_v, lse_o.at[pl.ds(base, RPW)])
    pltpu.sync_copy(t_v, tgl_o.at[pl.ds(base, RPW)])


@functools.partial(jax.jit, static_argnums=())
def _sc_call(table, idxf, tgtf):
    mesh = plsc.VectorSubcoreMesh(core_axis_name="c", subcore_axis_name="s")
    return pl.kernel(
        _sc_body,
        out_type=(
            jax.ShapeDtypeStruct((N_ROWS, D), jnp.float32),
            jax.ShapeDtypeStruct((N_ROWS,), jnp.float32),
            jax.ShapeDtypeStruct((N_ROWS,), jnp.float32),
        ),
        mesh=mesh,
        compiler_params=pltpu.CompilerParams(
            use_tc_tiling_on_sc=False, needs_layout_passes=False),
        scratch_types=[
            pltpu.VMEM((RPW,), jnp.int32),
            pltpu.VMEM((RPW,), jnp.int32),
            pltpu.VMEM((NBUF, CH, DP), jnp.float32),
            pltpu.VMEM((RPW,), jnp.float32),
            pltpu.VMEM((RPW,), jnp.float32),
            pltpu.SemaphoreType.DMA((NBUF,)),
            pltpu.SemaphoreType.DMA((NBUF,)),
        ],
    )(table, idxf, tgtf)


def _loss_body(l_ref, t_ref, o_ref):
    o_ref[0, 0] = jnp.sum(l_ref[...] - t_ref[...]) * (1.0 / N_ROWS)


def _loss_call(lse, tgtl):
    r = pl.pallas_call(
        _loss_body,
        out_shape=jax.ShapeDtypeStruct((1, 1), jnp.float32),
        out_specs=pl.BlockSpec(memory_space=pltpu.SMEM),
    )(lse.reshape(400, 128), tgtl.reshape(400, 128))
    return r[0, 0]


def kernel(idx, targets, table):
    idxf = idx.reshape(-1).astype(jnp.int32)
    tgtf = targets.reshape(-1).astype(jnp.int32)
    rowlse = _rowstats_call(table)
    ext = jnp.concatenate(
        [table, rowlse[:, None], jnp.zeros((D, DP - D - 1), jnp.float32)],
        axis=1)
    out, lse, tgtl = _sc_call(ext, idxf, tgtf)
    loss = _loss_call(lse, tgtl)
    return out, loss


# speedup vs baseline: 1.7147x; 1.0048x over previous
import functools

import jax
import jax.numpy as jnp
from jax import lax
from jax.experimental import pallas as pl
from jax.experimental.pallas import tpu as pltpu
from jax.experimental.pallas import tpu_sc as plsc

NC, NS, L = 2, 16, 16
NW = NC * NS
N_ROWS = 1024 * 50
D = 1000
DP = 1024
RPW = N_ROWS // NW
CH = 16
NCHUNK = RPW // CH
NBUF = 4


def _rowstats_body(tab_ref, lse_ref):
    x = tab_ref[...]
    m = jnp.max(x, axis=1, keepdims=True)
    s = jnp.sum(jnp.exp(x - m), axis=1, keepdims=True)
    lse_ref[...] = jnp.log(s) + m


def _rowstats_call(table):
    r = pl.pallas_call(
        _rowstats_body,
        out_shape=jax.ShapeDtypeStruct((D, 1), jnp.float32),
    )(table)
    return r.reshape(D)


def _sc_body(table, idxr, tgtr, out, lse_o, tgl_o,
             idx_v, tgt_v, rows_v, lse_v, t_v, gsem, osem):
    wid = lax.axis_index("s") * NC + lax.axis_index("c")
    base = wid * RPW
    pltpu.sync_copy(idxr.at[pl.ds(base, RPW)], idx_v)
    pltpu.sync_copy(tgtr.at[pl.ds(base, RPW)], tgt_v)

    lanes = lax.iota(jnp.int32, L)
    lse_col = jnp.full((L,), D, jnp.int32)

    def gather(c, b):
        iv = idx_v.at[pl.ds(c * CH, CH)]
        pltpu.make_async_copy(table.at[iv], rows_v.at[b], gsem.at[b]).start()

    gather(0, 0)
    gather(1, 1)

    def loop_body(i, carry):
        for j in range(NBUF):
            c = NBUF * i + j
            b, b2 = j, (j + 2) % NBUF
            iv = idx_v.at[pl.ds(c * CH, CH)]
            pltpu.make_async_copy(table.at[iv], rows_v.at[b], gsem.at[b]).wait()
            tg = tgt_v[pl.ds(c * CH, L)]
            t_v[pl.ds(c * CH, L)] = plsc.load_gather(rows_v.at[b], [lanes, tg])
            lse_v[pl.ds(c * CH, L)] = plsc.load_gather(
                rows_v.at[b], [lanes, lse_col])
            pltpu.make_async_copy(
                rows_v.at[b, :, pl.ds(0, D)],
                out.at[pl.ds(base + c * CH, CH)], osem.at[b]
            ).start()
            @pl.when(c >= 2)
            def _():
                pltpu.make_async_copy(
                    rows_v.at[b2, :, pl.ds(0, D)], out.at[pl.ds(base, CH)],
                    osem.at[b2]
                ).wait()

            @pl.when(c + 2 < NCHUNK)
            def _():
                gather(c + 2, b2)

        return carry

    lax.fori_loop(0, NCHUNK // NBUF, loop_body, 0)
    for b in ((NCHUNK - 2) % NBUF, (NCHUNK - 1) % NBUF):
        pltpu.make_async_copy(
            rows_v.at[b, :, pl.ds(0, D)], out.at[pl.ds(base, CH)], osem.at[b]
        ).wait()
    pltpu.sync_copy(lse_v, lse_o.at[pl.ds(base, RPW)])
    pltpu.sync_copy(t_v, tgl_o.at[pl.ds(base, RPW)])


@functools.partial(jax.jit, static_argnums=())
def _sc_call(table, idxf, tgtf):
    mesh = plsc.VectorSubcoreMesh(core_axis_name="c", subcore_axis_name="s")
    return pl.kernel(
        _sc_body,
        out_type=(
            jax.ShapeDtypeStruct((N_ROWS, D), jnp.float32),
            jax.ShapeDtypeStruct((N_ROWS,), jnp.float32),
            jax.ShapeDtypeStruct((N_ROWS,), jnp.float32),
        ),
        mesh=mesh,
        compiler_params=pltpu.CompilerParams(
            use_tc_tiling_on_sc=False, needs_layout_passes=False),
        scratch_types=[
            pltpu.VMEM((RPW,), jnp.int32),
            pltpu.VMEM((RPW,), jnp.int32),
            pltpu.VMEM((NBUF, CH, DP), jnp.float32),
            pltpu.VMEM((RPW,), jnp.float32),
            pltpu.VMEM((RPW,), jnp.float32),
            pltpu.SemaphoreType.DMA((NBUF,)),
            pltpu.SemaphoreType.DMA((NBUF,)),
        ],
    )(table, idxf, tgtf)


def _loss_body(l_ref, t_ref, o_ref):
    o_ref[0, 0] = jnp.sum(l_ref[...] - t_ref[...]) * (1.0 / N_ROWS)


def _loss_call(lse, tgtl):
    r = pl.pallas_call(
        _loss_body,
        out_shape=jax.ShapeDtypeStruct((1, 1), jnp.float32),
        out_specs=pl.BlockSpec(memory_space=pltpu.SMEM),
    )(lse.reshape(400, 128), tgtl.reshape(400, 128))
    return r[0, 0]


def kernel(idx, targets, table):
    idxf = idx.reshape(-1).astype(jnp.int32)
    tgtf = targets.reshape(-1).astype(jnp.int32)
    rowlse = _rowstats_call(table)
    ext = jnp.concatenate(
        [table, rowlse[:, None], jnp.zeros((D, DP - D - 1), jnp.float32)],
        axis=1)
    out, lse, tgtl = _sc_call(ext, idxf, tgtf)
    loss = _loss_call(lse, tgtl)
    return out, loss
